# Initial kernel scaffold; baseline (speedup 1.0000x reference)
import jax
import jax.numpy as jnp
from jax.experimental import pallas as pl


def kernel(x, edge_index, W1, b1, W2, b2):
    raise NotImplementedError("write your pallas kernel here")


# speedup vs baseline: 17.8916x; 17.8916x over previous
import functools

import jax
import jax.numpy as jnp
from jax import lax
from jax.experimental import pallas as pl
from jax.experimental.pallas import tpu as pltpu
from jax.experimental.pallas import tpu_sc as plsc

NC = 2
NS = 16
L = 16
NW = NC * NS
CH = 128


def _cdiv(a, b):
    return (a + b - 1) // b


def _sc_mesh():
    return plsc.VectorSubcoreMesh(core_axis_name="c", subcore_axis_name="s")


def _sc_degree(dst_p, n_pad, e_w):
    nsl = n_pad // NS
    n_chunks = e_w // CH

    @functools.partial(
        pl.kernel,
        out_type=jax.ShapeDtypeStruct((NC, NS, nsl), jnp.float32),
        mesh=_sc_mesh(),
        compiler_params=pltpu.CompilerParams(use_tc_tiling_on_sc=False, needs_layout_passes=False),
        scratch_types=[
            pltpu.VMEM((CH,), jnp.int32),
            pltpu.VMEM((CH,), jnp.float32),
            pltpu.VMEM((nsl,), jnp.float32),
            pltpu.VMEM_SHARED((n_pad,), jnp.float32),
        ],
    )
    def deg_kernel(dst_ref, out_ref, dst_v, pay_v, z_v, acc_sh):
        c = lax.axis_index("c")
        s = lax.axis_index("s")
        wid = c * NS + s
        zero16 = jnp.zeros((L,), jnp.float32)
        one16 = jnp.ones((L,), jnp.float32)

        def zb(i, carry):
            z_v[pl.ds(i * L, L)] = zero16
            return carry

        lax.fori_loop(0, nsl // L, zb, 0)

        def ob(i, carry):
            pay_v[pl.ds(i * L, L)] = one16
            return carry

        lax.fori_loop(0, CH // L, ob, 0)

        pltpu.sync_copy(z_v, acc_sh.at[pl.ds(s * nsl, nsl)])
        plsc.subcore_barrier()

        def chunk(j, carry):
            off = wid * e_w + j * CH
            pltpu.sync_copy(dst_ref.at[pl.ds(off, CH)], dst_v)
            pltpu.sync_copy(pay_v, acc_sh.at[dst_v], add=True)
            return carry

        lax.fori_loop(0, n_chunks, chunk, 0)
        plsc.subcore_barrier()
        pltpu.sync_copy(acc_sh.at[pl.ds(s * nsl, nsl)], out_ref.at[c, s])

    return deg_kernel(dst_p)


def _sc_scalar_wsum(src_p, dst_p, table, n_pad, e_w):
    nsl = n_pad // NS
    n_chunks = e_w // CH

    @functools.partial(
        pl.kernel,
        out_type=jax.ShapeDtypeStruct((NC, NS, nsl), jnp.float32),
        mesh=_sc_mesh(),
        compiler_params=pltpu.CompilerParams(use_tc_tiling_on_sc=False, needs_layout_passes=False),
        scratch_types=[
            pltpu.VMEM((CH,), jnp.int32),
            pltpu.VMEM((CH,), jnp.int32),
            pltpu.VMEM((CH,), jnp.float32),
            pltpu.VMEM((n_pad,), jnp.float32),
            pltpu.VMEM((nsl,), jnp.float32),
            pltpu.VMEM_SHARED((n_pad,), jnp.float32),
        ],
    )
    def wsum_kernel(src_ref, dst_ref, tab_ref, out_ref,
                    src_v, dst_v, pay_v, tab_v, z_v, acc_sh):
        c = lax.axis_index("c")
        s = lax.axis_index("s")
        wid = c * NS + s
        zero16 = jnp.zeros((L,), jnp.float32)

        def zb(i, carry):
            z_v[pl.ds(i * L, L)] = zero16
            return carry

        lax.fori_loop(0, nsl // L, zb, 0)
        pltpu.sync_copy(z_v, acc_sh.at[pl.ds(s * nsl, nsl)])
        pltpu.sync_copy(tab_ref, tab_v)
        plsc.subcore_barrier()

        def chunk(j, carry):
            off = wid * e_w + j * CH
            pltpu.sync_copy(src_ref.at[pl.ds(off, CH)], src_v)
            pltpu.sync_copy(dst_ref.at[pl.ds(off, CH)], dst_v)

            def g(kk, carry2):
                iv = src_v[pl.ds(kk * L, L)]
                pay_v[pl.ds(kk * L, L)] = plsc.load_gather(tab_v, [iv])
                return carry2

            lax.fori_loop(0, CH // L, g, 0)
            pltpu.sync_copy(pay_v, acc_sh.at[dst_v], add=True)
            return carry

        lax.fori_loop(0, n_chunks, chunk, 0)
        plsc.subcore_barrier()
        pltpu.sync_copy(acc_sh.at[pl.ds(s * nsl, nsl)], out_ref.at[c, s])

    return wsum_kernel(src_p, dst_p, table)


def _sc_row_wsum(src_p, dst_p, h1s_stack, n_pad, e_pad):
    nsl = n_pad // NS
    ect = e_pad // NS
    n_chunks = ect // CH
    zrows = nsl // 8

    @functools.partial(
        pl.kernel,
        out_type=jax.ShapeDtypeStruct((NC, NS, nsl, 32), jnp.float32),
        mesh=_sc_mesh(),
        compiler_params=pltpu.CompilerParams(use_tc_tiling_on_sc=False, needs_layout_passes=False),
        scratch_types=[
            pltpu.VMEM((CH,), jnp.int32),
            pltpu.VMEM((CH,), jnp.int32),
            pltpu.VMEM((CH, 32), jnp.float32),
            pltpu.VMEM((zrows, 32), jnp.float32),
            pltpu.VMEM_SHARED((n_pad, 32), jnp.float32),
        ],
    )
    def row_kernel(src_ref, dst_ref, tab_ref, out_ref,
                   src_v, dst_v, buf_v, zb_v, acc_sh):
        c = lax.axis_index("c")
        s = lax.axis_index("s")
        zero16 = jnp.zeros((L,), jnp.float32)

        def zb(r, carry):
            zb_v[r, pl.ds(0, L)] = zero16
            zb_v[r, pl.ds(L, L)] = zero16
            return carry

        lax.fori_loop(0, zrows, zb, 0)

        def zc(q, carry):
            pltpu.sync_copy(zb_v, acc_sh.at[pl.ds(s * nsl + q * zrows, zrows)])
            return carry

        lax.fori_loop(0, 8, zc, 0)
        plsc.subcore_barrier()

        def chunk(j, carry):
            off = s * ect + j * CH
            pltpu.sync_copy(src_ref.at[pl.ds(off, CH)], src_v)
            pltpu.sync_copy(dst_ref.at[pl.ds(off, CH)], dst_v)
            pltpu.sync_copy(tab_ref.at[c].at[src_v], buf_v)
            pltpu.sync_copy(buf_v, acc_sh.at[dst_v], add=True)
            return carry

        lax.fori_loop(0, n_chunks, chunk, 0)
        plsc.subcore_barrier()
        pltpu.sync_copy(acc_sh.at[pl.ds(s * nsl, nsl)], out_ref.at[c, s])

    return row_kernel(src_p, dst_p, h1s_stack)


def _tc_norm(degp, x2, n, r128):
    def body(degp_ref, x_ref, d_ref, xs_ref):
        deg = degp_ref[0] + degp_ref[1] + 1.0
        di = lax.rsqrt(deg)
        row = lax.broadcasted_iota(jnp.int32, deg.shape, 0)
        col = lax.broadcasted_iota(jnp.int32, deg.shape, 1)
        di = jnp.where(row * 128 + col < n, di, 0.0)
        d_ref[...] = di
        xs_ref[...] = x_ref[...] * di

    return pl.pallas_call(
        body,
        out_shape=(
            jax.ShapeDtypeStruct((r128, 128), jnp.float32),
            jax.ShapeDtypeStruct((r128, 128), jnp.float32),
        ),
    )(degp, x2)


def _tc_layer1(sp, dcol, xscol, w1, b1r, n_pad, rb):
    grid = n_pad // rb

    def body(sp_ref, d_ref, xs_ref, w1_ref, b1_ref, out_ref):
        s1 = sp_ref[0] + sp_ref[1]
        agg1 = d_ref[...] * (s1 + xs_ref[...])
        z = agg1 * w1_ref[...] + b1_ref[...]
        h1s = jnp.maximum(z, 0.0) * d_ref[...]
        out_ref[0] = h1s[:, :32]
        out_ref[1] = h1s[:, 32:]

    return pl.pallas_call(
        body,
        grid=(grid,),
        in_specs=[
            pl.BlockSpec((2, rb, 1), lambda i: (0, i, 0)),
            pl.BlockSpec((rb, 1), lambda i: (i, 0)),
            pl.BlockSpec((rb, 1), lambda i: (i, 0)),
            pl.BlockSpec((1, 64), lambda i: (0, 0)),
            pl.BlockSpec((1, 64), lambda i: (0, 0)),
        ],
        out_specs=pl.BlockSpec((2, rb, 32), lambda i: (0, i, 0)),
        out_shape=jax.ShapeDtypeStruct((2, n_pad, 32), jnp.float32),
    )(sp, dcol, xscol, w1, b1r)


def _tc_layer2(es, h1s, dcol, w2lo, w2hi, b2r, n_pad, rb):
    grid = n_pad // rb

    def body(es_ref, h_ref, d_ref, w2lo_ref, w2hi_ref, b2_ref, out_ref):
        d = d_ref[...]
        agg_lo = (es_ref[0] + h_ref[0]) * d
        agg_hi = (es_ref[1] + h_ref[1]) * d
        z = (
            jnp.dot(agg_lo, w2lo_ref[...], preferred_element_type=jnp.float32)
            + jnp.dot(agg_hi, w2hi_ref[...], preferred_element_type=jnp.float32)
            + b2_ref[...]
        )
        m = jnp.max(z, axis=1, keepdims=True)
        zz = z - m
        lse = jnp.log(jnp.sum(jnp.exp(zz), axis=1, keepdims=True))
        out_ref[...] = zz - lse

    return pl.pallas_call(
        body,
        grid=(grid,),
        in_specs=[
            pl.BlockSpec((2, rb, 32), lambda i: (0, i, 0)),
            pl.BlockSpec((2, rb, 32), lambda i: (0, i, 0)),
            pl.BlockSpec((rb, 1), lambda i: (i, 0)),
            pl.BlockSpec((32, 128), lambda i: (0, 0)),
            pl.BlockSpec((32, 128), lambda i: (0, 0)),
            pl.BlockSpec((1, 128), lambda i: (0, 0)),
        ],
        out_specs=pl.BlockSpec((rb, 128), lambda i: (i, 0)),
        out_shape=jax.ShapeDtypeStruct((n_pad, 128), jnp.float32),
    )(es, h1s, dcol, w2lo, w2hi, b2r)


def kernel(x, edge_index, W1, b1, W2, b2):
    n = x.shape[0]
    e = edge_index.shape[1]
    n_pad = _cdiv(n, 256) * 256
    r128 = n_pad // 128
    e_w = _cdiv(e, NW * CH) * CH
    e_pad = e_w * NW
    rb = n_pad // 8

    src = edge_index[0].astype(jnp.int32)
    dst = edge_index[1].astype(jnp.int32)
    npd = e_pad - e
    pad_src = (jnp.arange(npd, dtype=jnp.int32) * 97) % n
    pad_dst = n + jnp.arange(npd, dtype=jnp.int32) % (n_pad - n)
    src_p = jnp.concatenate([src, pad_src])
    dst_p = jnp.concatenate([dst, pad_dst])
    x2 = jnp.pad(x[:, 0], (0, n_pad - n)).reshape(r128, 128)

    degp = _sc_degree(dst_p, n_pad, e_w).reshape(2, r128, 128)
    d2, xs2 = _tc_norm(degp, x2, n, r128)
    dcol = d2.reshape(n_pad, 1)
    xs1 = xs2.reshape(n_pad)

    sp = _sc_scalar_wsum(src_p, dst_p, xs1, n_pad, e_w).reshape(2, n_pad, 1)
    h1s = _tc_layer1(sp, dcol, xs2.reshape(n_pad, 1), W1, b1.reshape(1, 64),
                     n_pad, rb)

    es = _sc_row_wsum(src_p, dst_p, h1s, n_pad, e_pad).reshape(2, n_pad, 32)

    out = _tc_layer2(es, h1s, dcol, W2[:32], W2[32:], b2.reshape(1, 128),
                     n_pad, rb)
    return out[:n]
